# baseline (speedup 1.0000x reference)
import functools

import jax
import jax.numpy as jnp
from jax import lax
from jax.experimental import pallas as pl
from jax.experimental.pallas import tpu as pltpu
from jax.experimental.pallas import tpu_sc as plsc

_NC = 2
_NS = 16
_LANES = 16
_DSLICES = 4
_CH = 64
_U = 16


def _sc_body(L, D, x4_hbm, p_hbm, out_hbm,
             xb0, xb1, ob0, ob1, pslab, qslab, ib0, ib1,
             gs0, gs1, ps, os0, os1):
    dw = D // _DSLICES
    nvec = dw // _LANES
    nch = L // _CH
    wid = lax.axis_index("s") * _NC + lax.axis_index("c")
    b = wid // _DSLICES
    dsl = wid % _DSLICES
    d0 = dsl * dw

    xbufs, obufs, ibufs = (xb0, xb1), (ob0, ob1), (ib0, ib1)
    gsems, osems = (gs0, gs1), (os0, os1)

    def o_copy(ci, par):
        return pltpu.make_async_copy(
            obufs[par], out_hbm.at[b, pl.ds(ci * _CH, _CH), pl.ds(d0, dw)],
            osems[par])

    def g_wait(par):
        pltpu.make_async_copy(
            x4_hbm.at[pl.ds(0, _U)], xbufs[par].at[pl.ds(0, _U)],
            gsems[par]).wait()

    def build_and_issue(ci, par):
        cnt = jnp.int32(0)
        for g in range(_CH // _LANES):
            t16 = ci * _CH + g * _LANES
            qv = qslab[pl.ds(t16, _LANES)]
            lanes = lax.iota(jnp.int32, _LANES)
            pos = lanes + t16
            vi = (b * L + pos) * _DSLICES + dsl
            c = jnp.int32(0)
            for r in range(_LANES):
                mr = qv[r] > 0.0
                idx_r = (b * L + (t16 + r)) * _DSLICES + dsl
                cm = jnp.where(mr, c, jnp.int32(-1))
                vi = jnp.where(lanes == cm, idx_r, vi)
                c = c + jnp.where(mr, jnp.int32(1), jnp.int32(0))
            ibufs[par][pl.ds(cnt, _LANES)] = vi
            cnt = cnt + c
        nu = (cnt + (_U - 1)) // _U

        def issue(u, _):
            vi = ibufs[par][pl.ds(u * _U, _U)]
            pltpu.make_async_copy(
                x4_hbm.at[vi], xbufs[par].at[pl.ds(u * _U, _U)],
                gsems[par]).start()
            return 0

        lax.fori_loop(0, nu, issue, 0)
        return nu

    pltpu.make_async_copy(p_hbm.at[b], pslab, ps).start()

    def zero_body(t, _):
        for j in range(nvec):
            xb0[t, pl.ds(j * _LANES, _LANES)] = jnp.zeros((_LANES,),
                                                          jnp.float32)
            xb1[t, pl.ds(j * _LANES, _LANES)] = jnp.zeros((_LANES,),
                                                          jnp.float32)
        return 0

    lax.fori_loop(0, _CH, zero_body, 0)
    for g in range(_CH // _LANES):
        ib0[pl.ds(g * _LANES, _LANES)] = jnp.zeros((_LANES,), jnp.int32)
        ib1[pl.ds(g * _LANES, _LANES)] = jnp.zeros((_LANES,), jnp.int32)

    pltpu.make_async_copy(p_hbm.at[b], pslab, ps).wait()

    def q_body(g, _):
        pv = pslab[pl.ds(g * _LANES, _LANES)]
        pos = lax.iota(jnp.int32, _LANES) + g * _LANES
        mask = (pv > 0.5) | (pos == 0)
        qslab[pl.ds(g * _LANES, _LANES)] = jnp.where(
            mask, jnp.clip(pv, 1e-4, 1.0 - 1e-4), 0.0)
        return 0

    lax.fori_loop(0, L // _LANES, q_body, 0)

    nu0 = build_and_issue(0, 0)

    def pair_body(cp, carry):
        h, nu_cur = carry[:-1], carry[-1]
        for par in (0, 1):
            ci = 2 * cp + par
            cin = jnp.minimum(ci + 1, nch - 1)
            nu_next = build_and_issue(cin, 1 - par)

            def wbody(u, _):
                g_wait(par)
                return 0

            lax.fori_loop(0, nu_cur, wbody, 0)
            xbuf, obuf = xbufs[par], obufs[par]

            @pl.when(ci >= 2)
            def _drain():
                o_copy(ci - 2, par).wait()

            def group_body(g, hsk):
                hs, k = list(hsk[:-1]), hsk[-1]
                qv = qslab[pl.ds(ci * _CH + g * _LANES, _LANES)]
                for r in range(_LANES):
                    qt = qv[r]
                    t = g * _LANES + r
                    for j in range(nvec):
                        xv = xbuf[k, pl.ds(j * _LANES, _LANES)]
                        hs[j] = hs[j] + qt * (xv - hs[j])
                        obuf[t, pl.ds(j * _LANES, _LANES)] = hs[j]
                    k = k + (qt > 0.0).astype(jnp.int32)
                return tuple(hs) + (k,)

            hk = lax.fori_loop(0, _CH // _LANES, group_body,
                               tuple(h) + (jnp.int32(0),))
            h = hk[:-1]
            o_copy(ci, par).start()
            nu_cur = nu_next
        return tuple(h) + (nu_cur,)

    h0 = tuple(jnp.zeros((_LANES,), jnp.float32) for _ in range(nvec))
    final = lax.fori_loop(0, nch // 2, pair_body, h0 + (nu0,))
    def fdrain(u, _):
        g_wait(0)
        return 0

    lax.fori_loop(0, final[-1], fdrain, 0)
    o_copy(nch - 2, 0).wait()
    o_copy(nch - 1, 1).wait()


def kernel(hidden_states, boundary_prob):
    B, L, D = hidden_states.shape
    dw = D // _DSLICES
    x4 = hidden_states.reshape(B * L * _DSLICES, dw)
    mesh = plsc.VectorSubcoreMesh(core_axis_name="c", subcore_axis_name="s")
    k = functools.partial(
        pl.kernel,
        mesh=mesh,
        out_type=jax.ShapeDtypeStruct((B, L, D), jnp.float32),
        scratch_types=[
            pltpu.VMEM((_CH, dw), jnp.float32),
            pltpu.VMEM((_CH, dw), jnp.float32),
            pltpu.VMEM((_CH, dw), jnp.float32),
            pltpu.VMEM((_CH, dw), jnp.float32),
            pltpu.VMEM((L,), jnp.float32),
            pltpu.VMEM((L,), jnp.float32),
            pltpu.VMEM((_CH,), jnp.int32),
            pltpu.VMEM((_CH,), jnp.int32),
            pltpu.SemaphoreType.DMA,
            pltpu.SemaphoreType.DMA,
            pltpu.SemaphoreType.DMA,
            pltpu.SemaphoreType.DMA,
            pltpu.SemaphoreType.DMA,
        ],
    )(functools.partial(_sc_body, L, D))
    return k(x4, boundary_prob)


# speedup vs baseline: 5.7552x; 5.7552x over previous
import functools

import jax
import jax.numpy as jnp
from jax import lax
from jax.experimental import pallas as pl
from jax.experimental.pallas import tpu as pltpu
from jax.experimental.pallas import tpu_sc as plsc

_NC = 2
_NS = 16
_LANES = 16
_DSLICES = 4
_CH = 64


def _sc_body(x_hbm, p_hbm, out_hbm,
             xb0, xb1, ob0, ob1, ob2, ob3, pslab, qslab,
             xs0, xs1, ps, os0, os1, os2, os3):
    B, L, D = x_hbm.shape
    dw = D // _DSLICES
    nvec = dw // _LANES
    nch = L // _CH
    wid = lax.axis_index("s") * _NC + lax.axis_index("c")
    b = wid // _DSLICES
    d0 = (wid % _DSLICES) * dw

    xbufs, obufs = (xb0, xb1), (ob0, ob1, ob2, ob3)
    xsems, osems = (xs0, xs1), (os0, os1, os2, os3)

    def x_copy(ci, par):
        return pltpu.make_async_copy(
            x_hbm.at[b, pl.ds(ci * _CH, _CH), pl.ds(d0, dw)],
            xbufs[par], xsems[par])

    def o_copy(ci, par):
        return pltpu.make_async_copy(
            obufs[par], out_hbm.at[b, pl.ds(ci * _CH, _CH), pl.ds(d0, dw)],
            osems[par])

    pltpu.make_async_copy(p_hbm.at[b], pslab, ps).start()
    x_copy(0, 0).start()
    pltpu.make_async_copy(p_hbm.at[b], pslab, ps).wait()

    def q_body(g, _):
        pv = pslab[pl.ds(g * _LANES, _LANES)]
        pos = lax.iota(jnp.int32, _LANES) + g * _LANES
        mask = (pv > 0.5) | (pos == 0)
        qslab[pl.ds(g * _LANES, _LANES)] = jnp.where(
            mask, jnp.clip(pv, 1e-4, 1.0 - 1e-4), 0.0)
        return 0

    lax.fori_loop(0, L // _LANES, q_body, 0)

    def quad_body(cp, h):
        for par in (0, 1, 2, 3):
            ci = 4 * cp + par
            xpar = par % 2
            @pl.when(ci + 1 < nch)
            def _pref():
                x_copy(ci + 1, 1 - xpar).start()

            x_copy(ci, xpar).wait()
            xbuf, obuf = xbufs[xpar], obufs[par]

            @pl.when(ci >= 4)
            def _drain():
                o_copy(ci - 4, par).wait()

            def group_body(g, hs):
                qv = qslab[pl.ds(ci * _CH + g * _LANES, _LANES)]
                hs = list(hs)
                for r in range(_LANES):
                    qt = qv[r]
                    t = g * _LANES + r
                    for j in range(nvec):
                        xv = xbuf[t, pl.ds(j * _LANES, _LANES)]
                        hs[j] = hs[j] + qt * (xv - hs[j])
                        obuf[t, pl.ds(j * _LANES, _LANES)] = hs[j]
                return tuple(hs)

            h = lax.fori_loop(0, _CH // _LANES, group_body, h)
            o_copy(ci, par).start()
        return h

    h0 = tuple(jnp.zeros((_LANES,), jnp.float32) for _ in range(nvec))
    lax.fori_loop(0, nch // 4, quad_body, h0)
    o_copy(nch - 4, 0).wait()
    o_copy(nch - 3, 1).wait()
    o_copy(nch - 2, 2).wait()
    o_copy(nch - 1, 3).wait()


def kernel(hidden_states, boundary_prob):
    B, L, D = hidden_states.shape
    dw = D // _DSLICES
    mesh = plsc.VectorSubcoreMesh(core_axis_name="c", subcore_axis_name="s")
    k = functools.partial(
        pl.kernel,
        mesh=mesh,
        out_type=jax.ShapeDtypeStruct((B, L, D), jnp.float32),
        scratch_types=[
            pltpu.VMEM((_CH, dw), jnp.float32),
            pltpu.VMEM((_CH, dw), jnp.float32),
            pltpu.VMEM((_CH, dw), jnp.float32),
            pltpu.VMEM((_CH, dw), jnp.float32),
            pltpu.VMEM((_CH, dw), jnp.float32),
            pltpu.VMEM((_CH, dw), jnp.float32),
            pltpu.VMEM((L,), jnp.float32),
            pltpu.VMEM((L,), jnp.float32),
            pltpu.SemaphoreType.DMA,
            pltpu.SemaphoreType.DMA,
            pltpu.SemaphoreType.DMA,
            pltpu.SemaphoreType.DMA,
            pltpu.SemaphoreType.DMA,
            pltpu.SemaphoreType.DMA,
            pltpu.SemaphoreType.DMA,
        ],
    )(_sc_body)
    return k(hidden_states, boundary_prob)


# speedup vs baseline: 5.8954x; 1.0244x over previous
import functools

import jax
import jax.numpy as jnp
from jax import lax
from jax.experimental import pallas as pl
from jax.experimental.pallas import tpu as pltpu
from jax.experimental.pallas import tpu_sc as plsc

_NC = 2
_NS = 16
_LANES = 16
_DSLICES = 4
_CH = 64


def _sc_body(x_hbm, p_hbm, out_hbm,
             xb0, xb1, ob0, ob1, pslab, qslab,
             xs0, xs1, ps, os0, os1):
    B, L, D = x_hbm.shape
    dw = D // _DSLICES
    nvec = dw // _LANES
    nch = L // _CH
    wid = lax.axis_index("s") * _NC + lax.axis_index("c")
    b = wid // _DSLICES
    d0 = (wid % _DSLICES) * dw

    xbufs, obufs = (xb0, xb1), (ob0, ob1)
    xsems, osems = (xs0, xs1), (os0, os1)

    def x_copy(ci, par):
        return pltpu.make_async_copy(
            x_hbm.at[b, pl.ds(ci * _CH, _CH), pl.ds(d0, dw)],
            xbufs[par], xsems[par])

    def o_copy(ci, par):
        return pltpu.make_async_copy(
            obufs[par], out_hbm.at[b, pl.ds(ci * _CH, _CH), pl.ds(d0, dw)],
            osems[par])

    pltpu.make_async_copy(p_hbm.at[b], pslab, ps).start()
    x_copy(0, 0).start()
    pltpu.make_async_copy(p_hbm.at[b], pslab, ps).wait()

    def q_body(g, _):
        pv = pslab[pl.ds(g * _LANES, _LANES)]
        pos = lax.iota(jnp.int32, _LANES) + g * _LANES
        mask = (pv > 0.5) | (pos == 0)
        qslab[pl.ds(g * _LANES, _LANES)] = jnp.where(
            mask, jnp.clip(pv, 1e-4, 1.0 - 1e-4), 0.0)
        return 0

    lax.fori_loop(0, L // _LANES, q_body, 0)

    def pair_body(cp, h):
        for par in (0, 1):
            ci = 2 * cp + par
            @pl.when(ci + 1 < nch)
            def _pref():
                x_copy(ci + 1, 1 - par).start()

            x_copy(ci, par).wait()
            xbuf, obuf = xbufs[par], obufs[par]

            @pl.when(ci >= 2)
            def _drain():
                o_copy(ci - 2, par).wait()

            def group_body(g, hs):
                qv = qslab[pl.ds(ci * _CH + g * _LANES, _LANES)]
                hs = list(hs)
                for r in range(_LANES):
                    qt = qv[r]
                    t = g * _LANES + r
                    for j in range(nvec):
                        xv = xbuf[t, pl.ds(j * _LANES, _LANES)]
                        hs[j] = hs[j] + qt * (xv - hs[j])
                        obuf[t, pl.ds(j * _LANES, _LANES)] = hs[j]
                return tuple(hs)

            h = lax.fori_loop(0, _CH // _LANES, group_body, h)
            o_copy(ci, par).start()
        return h

    h0 = tuple(jnp.zeros((_LANES,), jnp.float32) for _ in range(nvec))
    lax.fori_loop(0, nch // 2, pair_body, h0)
    o_copy(nch - 2, 0).wait()
    o_copy(nch - 1, 1).wait()


def kernel(hidden_states, boundary_prob):
    B, L, D = hidden_states.shape
    dw = D // _DSLICES
    mesh = plsc.VectorSubcoreMesh(core_axis_name="c", subcore_axis_name="s")
    k = functools.partial(
        pl.kernel,
        mesh=mesh,
        out_type=jax.ShapeDtypeStruct((B, L, D), jnp.float32),
        scratch_types=[
            pltpu.VMEM((_CH, dw), jnp.float32),
            pltpu.VMEM((_CH, dw), jnp.float32),
            pltpu.VMEM((_CH, dw), jnp.float32),
            pltpu.VMEM((_CH, dw), jnp.float32),
            pltpu.VMEM((L,), jnp.float32),
            pltpu.VMEM((L,), jnp.float32),
            pltpu.SemaphoreType.DMA,
            pltpu.SemaphoreType.DMA,
            pltpu.SemaphoreType.DMA,
            pltpu.SemaphoreType.DMA,
            pltpu.SemaphoreType.DMA,
        ],
    )(_sc_body)
    return k(hidden_states, boundary_prob)
